# Initial kernel scaffold; baseline (speedup 1.0000x reference)
import jax
import jax.numpy as jnp
from jax.experimental import pallas as pl


def kernel(word_ids, age_ids, word_table, age_table, gamma, beta):
    raise NotImplementedError("write your pallas kernel here")


# speedup vs baseline: 2.7813x; 2.7813x over previous
import functools

import jax
import jax.numpy as jnp
from jax import lax
from jax.experimental import pallas as pl
from jax.experimental.pallas import tpu as pltpu
from jax.experimental.pallas import tpu_sc as plsc

HIDDEN = 64
EPS = 1e-12

def _make_sc_gather(rows, chunk):
    info = plsc.get_sparse_core_info()
    nc, nw = info.num_cores, info.num_cores * info.num_subcores
    rpw = rows // nw
    nchunk = rpw // chunk
    mesh = plsc.VectorSubcoreMesh(core_axis_name="c", subcore_axis_name="s")

    @functools.partial(
        pl.kernel,
        mesh=mesh,
        compiler_params=pltpu.CompilerParams(use_tc_tiling_on_sc=False),
        out_type=jax.ShapeDtypeStruct((rows, HIDDEN), jnp.float32),
        scratch_types=[
            pltpu.VMEM((chunk,), jnp.int32),
            pltpu.VMEM((chunk, HIDDEN), jnp.float32),
            pltpu.SemaphoreType.DMA,
        ],
    )
    def sc_gather(idx_hbm, tab_hbm, out_hbm, idx_v, rows_v, sem):
        w = lax.axis_index("s") * nc + lax.axis_index("c")

        def body(i, carry):
            base = w * rpw + i * chunk
            pltpu.sync_copy(idx_hbm.at[pl.ds(base, chunk)], idx_v)
            pltpu.async_copy(tab_hbm.at[idx_v], rows_v, sem).wait()
            pltpu.sync_copy(rows_v, out_hbm.at[pl.ds(base, chunk)])
            return carry

        lax.fori_loop(0, nchunk, body, 0)

    return sc_gather


def _ln_body(aid_ref, rows_ref, atab_ref, g_ref, b_ref, out_ref, *, blk):
    ids = aid_ref[0, 0, :]
    onehot = (ids[:, None] == lax.broadcasted_iota(jnp.int32, (blk, 128), 1))
    age = jnp.dot(onehot.astype(jnp.float32), atab_ref[...],
                  preferred_element_type=jnp.float32)
    x = rows_ref[...] + age
    u = jnp.mean(x, axis=-1, keepdims=True)
    s = jnp.mean((x - u) ** 2, axis=-1, keepdims=True)
    xn = (x - u) * lax.rsqrt(s + EPS)
    out_ref[...] = g_ref[...] * xn + b_ref[...]


def _make_tc_ln(rows, blk):
    grid = rows // blk
    return pl.pallas_call(
        functools.partial(_ln_body, blk=blk),
        grid=(grid,),
        in_specs=[
            pl.BlockSpec((1, 1, blk), lambda i: (i, 0, 0)),
            pl.BlockSpec((blk, HIDDEN), lambda i: (i, 0)),
            pl.BlockSpec((128, HIDDEN), lambda i: (0, 0)),
            pl.BlockSpec((1, HIDDEN), lambda i: (0, 0)),
            pl.BlockSpec((1, HIDDEN), lambda i: (0, 0)),
        ],
        out_specs=pl.BlockSpec((blk, HIDDEN), lambda i: (i, 0)),
        out_shape=jax.ShapeDtypeStruct((rows, HIDDEN), jnp.float32),
    )


def kernel(word_ids, age_ids, word_table, age_table, gamma, beta):
    b, l = word_ids.shape
    rows = b * l
    wids = word_ids.reshape(rows)
    summed = _make_sc_gather(rows, 128)(wids, word_table)

    blk = 512
    aids = age_ids.reshape(rows // blk, 1, blk)
    out = _make_tc_ln(rows, blk)(
        aids, summed, age_table,
        gamma.reshape(1, HIDDEN), beta.reshape(1, HIDDEN))
    return out.reshape(b, l, HIDDEN)


# speedup vs baseline: 4.4572x; 1.6026x over previous
import functools

import jax
import jax.numpy as jnp
from jax import lax
from jax.experimental import pallas as pl
from jax.experimental.pallas import tpu as pltpu
from jax.experimental.pallas import tpu_sc as plsc

HIDDEN = 64
EPS = 1e-12
SB = 800
CP = 80


def _make_sc_gather(pairs):
    info = plsc.get_sparse_core_info()
    nc, nw = info.num_cores, info.num_cores * info.num_subcores
    rpw = pairs // nw
    nchunk = rpw // CP
    mesh = plsc.VectorSubcoreMesh(core_axis_name="c", subcore_axis_name="s")

    @functools.partial(
        pl.kernel,
        mesh=mesh,
        compiler_params=pltpu.CompilerParams(use_tc_tiling_on_sc=False),
        out_type=jax.ShapeDtypeStruct((pairs, 2 * HIDDEN), jnp.float32),
        scratch_types=[
            pltpu.VMEM((CP,), jnp.int32),
            pltpu.VMEM((CP,), jnp.int32),
            pltpu.VMEM((CP, HIDDEN), jnp.float32),
            pltpu.VMEM((CP, HIDDEN), jnp.float32),
            pltpu.SemaphoreType.DMA,
            pltpu.SemaphoreType.DMA,
        ],
    )
    def sc_gather(idx_hbm, tab_hbm, out_hbm, idxl_v, idxr_v, rl_v, rr_v,
                  seml, semr):
        w = lax.axis_index("s") * nc + lax.axis_index("c")

        def body(c, carry):
            p = w * rpw + c * CP
            sb = p // SB
            lb = sb * (2 * SB) + (p - sb * SB)
            pltpu.sync_copy(idx_hbm.at[pl.ds(lb, CP)], idxl_v)
            pltpu.sync_copy(idx_hbm.at[pl.ds(lb + SB, CP)], idxr_v)
            cl = pltpu.async_copy(tab_hbm.at[idxl_v], rl_v, seml)
            cr = pltpu.async_copy(tab_hbm.at[idxr_v], rr_v, semr)
            cl.wait()
            cr.wait()
            pltpu.sync_copy(rl_v, out_hbm.at[pl.ds(p, CP), pl.ds(0, HIDDEN)])
            pltpu.sync_copy(rr_v,
                            out_hbm.at[pl.ds(p, CP), pl.ds(HIDDEN, HIDDEN)])
            return carry

        lax.fori_loop(0, nchunk, body, 0)

    return sc_gather


def _ln_body(aid_ref, rows_ref, atab_ref, g_ref, b_ref, out_ref):
    x = rows_ref[...]
    ids = aid_ref[0, 0, :]
    atab = atab_ref[...]
    g = g_ref[...]
    b = b_ref[...]
    for h in range(2):
        idh = ids[h * SB:(h + 1) * SB]
        onehot = (idh[:, None] ==
                  lax.broadcasted_iota(jnp.int32, (SB, 128), 1))
        age = jnp.dot(onehot.astype(jnp.float32), atab,
                      preferred_element_type=jnp.float32)
        xh = x[:, h * HIDDEN:(h + 1) * HIDDEN] + age
        u = jnp.mean(xh, axis=-1, keepdims=True)
        s = jnp.mean((xh - u) ** 2, axis=-1, keepdims=True)
        out_ref[pl.ds(h * SB, SB), :] = g * ((xh - u) * lax.rsqrt(s + EPS)) + b


def _make_tc_ln(rows):
    grid = rows // (2 * SB)
    return pl.pallas_call(
        _ln_body,
        grid=(grid,),
        in_specs=[
            pl.BlockSpec((1, 1, 2 * SB), lambda i: (i, 0, 0)),
            pl.BlockSpec((SB, 2 * HIDDEN), lambda i: (i, 0)),
            pl.BlockSpec((128, HIDDEN), lambda i: (0, 0)),
            pl.BlockSpec((1, HIDDEN), lambda i: (0, 0)),
            pl.BlockSpec((1, HIDDEN), lambda i: (0, 0)),
        ],
        out_specs=pl.BlockSpec((2 * SB, HIDDEN), lambda i: (i, 0)),
        out_shape=jax.ShapeDtypeStruct((rows, HIDDEN), jnp.float32),
    )


def kernel(word_ids, age_ids, word_table, age_table, gamma, beta):
    b, l = word_ids.shape
    rows = b * l
    wids = word_ids.reshape(rows)
    packed = _make_sc_gather(rows // 2)(wids, word_table)

    aids = age_ids.reshape(rows // (2 * SB), 1, 2 * SB)
    out = _make_tc_ln(rows)(
        aids, packed, age_table,
        gamma.reshape(1, HIDDEN), beta.reshape(1, HIDDEN))
    return out.reshape(b, l, HIDDEN)


# speedup vs baseline: 4.6009x; 1.0322x over previous
import functools

import jax
import jax.numpy as jnp
from jax import lax
from jax.experimental import pallas as pl
from jax.experimental.pallas import tpu as pltpu
from jax.experimental.pallas import tpu_sc as plsc

HIDDEN = 64
EPS = 1e-12
SB = 800
CP = 80


def _make_sc_gather(pairs):
    info = plsc.get_sparse_core_info()
    nc, nw = info.num_cores, info.num_cores * info.num_subcores
    rpw = pairs // nw
    nchunk = rpw // CP
    mesh = plsc.VectorSubcoreMesh(core_axis_name="c", subcore_axis_name="s")

    @functools.partial(
        pl.kernel,
        mesh=mesh,
        compiler_params=pltpu.CompilerParams(use_tc_tiling_on_sc=False),
        out_type=jax.ShapeDtypeStruct((pairs, 2 * HIDDEN), jnp.float32),
        scratch_types=[
            pltpu.VMEM((CP,), jnp.int32),
            pltpu.VMEM((CP,), jnp.int32),
            pltpu.VMEM((CP, HIDDEN), jnp.float32),
            pltpu.VMEM((CP, HIDDEN), jnp.float32),
            pltpu.SemaphoreType.DMA,
            pltpu.SemaphoreType.DMA,
        ],
    )
    def sc_gather(idx_hbm, tab_hbm, out_hbm, idxl_v, idxr_v, rl_v, rr_v,
                  seml, semr):
        w = lax.axis_index("s") * nc + lax.axis_index("c")

        def body(c, carry):
            p = w * rpw + c * CP
            sb = p // SB
            lb = sb * (2 * SB) + (p - sb * SB)
            pltpu.sync_copy(idx_hbm.at[pl.ds(lb, CP)], idxl_v)
            pltpu.sync_copy(idx_hbm.at[pl.ds(lb + SB, CP)], idxr_v)
            cl = pltpu.async_copy(tab_hbm.at[idxl_v], rl_v, seml)
            cr = pltpu.async_copy(tab_hbm.at[idxr_v], rr_v, semr)
            cl.wait()
            cr.wait()
            pltpu.sync_copy(rl_v, out_hbm.at[pl.ds(p, CP), pl.ds(0, HIDDEN)])
            pltpu.sync_copy(rr_v,
                            out_hbm.at[pl.ds(p, CP), pl.ds(HIDDEN, HIDDEN)])
            return carry

        lax.fori_loop(0, nchunk, body, 0)

    return sc_gather


def _ln_body(aid_ref, rows_ref, atab2_ref, bd_ref, g_ref, b_ref, out_ref):
    x = rows_ref[...]
    ids = aid_ref[0, 0, :]
    oh = [
        (ids[h * SB:(h + 1) * SB][:, None] ==
         lax.broadcasted_iota(jnp.int32, (SB, 128), 1)).astype(jnp.float32)
        for h in range(2)
    ]
    age = (jnp.dot(oh[0], atab2_ref[0], preferred_element_type=jnp.float32) +
           jnp.dot(oh[1], atab2_ref[1], preferred_element_type=jnp.float32))
    xh = x + age
    bd = bd_ref[...]
    u = jnp.dot(xh, bd, preferred_element_type=jnp.float32)
    d = xh - u
    s = jnp.dot(d * d, bd, preferred_element_type=jnp.float32)
    y = g_ref[...] * (d * lax.rsqrt(s + EPS)) + b_ref[...]
    y3 = jnp.concatenate([y[:, :HIDDEN], y[:, HIDDEN:]], axis=0)
    nb = (2 * SB) // 200
    out_ref[...] = y3.reshape(nb, 200, HIDDEN)


def _make_tc_ln(b, l):
    rows = b * l
    grid = rows // (2 * SB)
    nb = (2 * SB) // l if l <= 2 * SB else 0
    return pl.pallas_call(
        _ln_body,
        grid=(grid,),
        in_specs=[
            pl.BlockSpec((1, 1, 2 * SB), lambda i: (i, 0, 0)),
            pl.BlockSpec((SB, 2 * HIDDEN), lambda i: (i, 0)),
            pl.BlockSpec((2, 128, 128), lambda i: (0, 0, 0)),
            pl.BlockSpec((128, 128), lambda i: (0, 0)),
            pl.BlockSpec((1, 2 * HIDDEN), lambda i: (0, 0)),
            pl.BlockSpec((1, 2 * HIDDEN), lambda i: (0, 0)),
        ],
        out_specs=pl.BlockSpec((nb, l, HIDDEN), lambda i: (i, 0, 0)),
        out_shape=jax.ShapeDtypeStruct((b, l, HIDDEN), jnp.float32),
    )


def kernel(word_ids, age_ids, word_table, age_table, gamma, beta):
    b, l = word_ids.shape
    rows = b * l
    wids = word_ids.reshape(rows)
    packed = _make_sc_gather(rows // 2)(wids, word_table)

    aids = age_ids.reshape(rows // (2 * SB), 1, 2 * SB)
    atab2 = jnp.zeros((2, 128, 128), jnp.float32)
    atab2 = atab2.at[0, :, :HIDDEN].set(age_table)
    atab2 = atab2.at[1, :, HIDDEN:].set(age_table)
    bd = jnp.zeros((128, 128), jnp.float32)
    bd = bd.at[:HIDDEN, :HIDDEN].set(1.0 / HIDDEN)
    bd = bd.at[HIDDEN:, HIDDEN:].set(1.0 / HIDDEN)
    g2 = jnp.concatenate([gamma, gamma]).reshape(1, 2 * HIDDEN)
    b2 = jnp.concatenate([beta, beta]).reshape(1, 2 * HIDDEN)
    return _make_tc_ln(b, l)(aids, packed, atab2, bd, g2, b2)


# speedup vs baseline: 4.9926x; 1.0851x over previous
import functools

import jax
import jax.numpy as jnp
from jax import lax
from jax.experimental import pallas as pl
from jax.experimental.pallas import tpu as pltpu
from jax.experimental.pallas import tpu_sc as plsc

HIDDEN = 64
EPS = 1e-12
SB = 800
CP = 80


def _make_sc_gather(pairs):
    info = plsc.get_sparse_core_info()
    nc, nw = info.num_cores, info.num_cores * info.num_subcores
    rpw = pairs // nw
    nchunk = rpw // CP
    mesh = plsc.VectorSubcoreMesh(core_axis_name="c", subcore_axis_name="s")

    @functools.partial(
        pl.kernel,
        mesh=mesh,
        compiler_params=pltpu.CompilerParams(use_tc_tiling_on_sc=False),
        out_type=jax.ShapeDtypeStruct((pairs, 2 * HIDDEN), jnp.float32),
        scratch_types=[
            pltpu.VMEM((CP,), jnp.int32),
            pltpu.VMEM((CP,), jnp.int32),
            pltpu.VMEM((CP, HIDDEN), jnp.float32),
            pltpu.VMEM((CP, HIDDEN), jnp.float32),
            pltpu.SemaphoreType.DMA,
            pltpu.SemaphoreType.DMA,
        ],
    )
    def sc_gather(idx_hbm, tab_hbm, out_hbm, idxl_v, idxr_v, rl_v, rr_v,
                  seml, semr):
        w = lax.axis_index("s") * nc + lax.axis_index("c")

        def body(c, carry):
            p = w * rpw + c * CP
            sb = p // SB
            lb = sb * (2 * SB) + (p - sb * SB)
            pltpu.sync_copy(idx_hbm.at[pl.ds(lb, CP)], idxl_v)
            pltpu.sync_copy(idx_hbm.at[pl.ds(lb + SB, CP)], idxr_v)
            cl = pltpu.async_copy(tab_hbm.at[idxl_v], rl_v, seml)
            cr = pltpu.async_copy(tab_hbm.at[idxr_v], rr_v, semr)
            cl.wait()
            cr.wait()
            pltpu.sync_copy(rl_v, out_hbm.at[pl.ds(p, CP), pl.ds(0, HIDDEN)])
            pltpu.sync_copy(rr_v,
                            out_hbm.at[pl.ds(p, CP), pl.ds(HIDDEN, HIDDEN)])
            return carry

        lax.fori_loop(0, nchunk, body, 0)

    return sc_gather


def _ln_body(aid_ref, rows_ref, atab2_ref, bd_ref, g_ref, b_ref, out_ref):
    x = rows_ref[...]
    ids = aid_ref[0, 0, :]
    oh = [
        (ids[h * SB:(h + 1) * SB][:, None] ==
         lax.broadcasted_iota(jnp.int32, (SB, 128), 1)).astype(jnp.float32)
        for h in range(2)
    ]
    age = (jnp.dot(oh[0], atab2_ref[0], preferred_element_type=jnp.float32) +
           jnp.dot(oh[1], atab2_ref[1], preferred_element_type=jnp.float32))
    xh = x + age
    bd = bd_ref[...]
    u = jnp.dot(xh, bd, preferred_element_type=jnp.float32)
    d = xh - u
    s = jnp.dot(d * d, bd, preferred_element_type=jnp.float32)
    y = g_ref[...] * (d * lax.rsqrt(s + EPS)) + b_ref[...]
    out_ref[...] = jnp.concatenate([y[:, :HIDDEN], y[:, HIDDEN:]], axis=0)


def _make_tc_ln(rows):
    grid = rows // (2 * SB)
    return pl.pallas_call(
        _ln_body,
        grid=(grid,),
        in_specs=[
            pl.BlockSpec((1, 1, 2 * SB), lambda i: (i, 0, 0)),
            pl.BlockSpec((SB, 2 * HIDDEN), lambda i: (i, 0)),
            pl.BlockSpec((2, 128, 128), lambda i: (0, 0, 0)),
            pl.BlockSpec((128, 128), lambda i: (0, 0)),
            pl.BlockSpec((1, 2 * HIDDEN), lambda i: (0, 0)),
            pl.BlockSpec((1, 2 * HIDDEN), lambda i: (0, 0)),
        ],
        out_specs=pl.BlockSpec((2 * SB, HIDDEN), lambda i: (i, 0)),
        out_shape=jax.ShapeDtypeStruct((rows, HIDDEN), jnp.float32),
    )


def kernel(word_ids, age_ids, word_table, age_table, gamma, beta):
    b, l = word_ids.shape
    rows = b * l
    wids = word_ids.reshape(rows)
    packed = _make_sc_gather(rows // 2)(wids, word_table)

    aids = age_ids.reshape(rows // (2 * SB), 1, 2 * SB)
    atab2 = jnp.zeros((2, 128, 128), jnp.float32)
    atab2 = atab2.at[0, :, :HIDDEN].set(age_table)
    atab2 = atab2.at[1, :, HIDDEN:].set(age_table)
    bd = jnp.zeros((128, 128), jnp.float32)
    bd = bd.at[:HIDDEN, :HIDDEN].set(1.0 / HIDDEN)
    bd = bd.at[HIDDEN:, HIDDEN:].set(1.0 / HIDDEN)
    g2 = jnp.concatenate([gamma, gamma]).reshape(1, 2 * HIDDEN)
    b2 = jnp.concatenate([beta, beta]).reshape(1, 2 * HIDDEN)
    out = _make_tc_ln(rows)(aids, packed, atab2, bd, g2, b2)
    return out.reshape(b, l, HIDDEN)
